# Initial kernel scaffold; baseline (speedup 1.0000x reference)
import jax
import jax.numpy as jnp
from jax.experimental import pallas as pl


def kernel(x, edge_index, batch, sigma, W1, b1, W2, b2, W3, b3, g1, be1, g2, be2, g3, be3, Wf1, bf1, Wf2, bf2, Wfc, bfc):
    raise NotImplementedError("write your pallas kernel here")


# speedup vs baseline: 15.7980x; 15.7980x over previous
import functools

import jax
import jax.numpy as jnp
from jax import lax
from jax.experimental import pallas as pl
from jax.experimental.pallas import tpu as pltpu
from jax.experimental.pallas import tpu_sc as plsc

_N = 10000
_E = 320000
_D = 128
_H = 64
_G = 16

_NC = 2
_NS = 16
_NW = _NC * _NS
_CHUNK = 128
_NCHUNK = _E // _CHUNK
_RPT = 632
_NP = _RPT * _NS
_DEGW = 16

_mesh = plsc.VectorSubcoreMesh(
    core_axis_name="c", subcore_axis_name="s", num_cores=_NC, num_subcores=_NS
)


def _zero_fill(ref, rows, width):
  zv = jnp.zeros((16,), jnp.float32)
  nw = width // 16

  def body(i, carry):
    r = i // nw
    cs = (i % nw) * 16
    ref[r, pl.ds(cs, 16)] = zv
    return carry

  lax.fori_loop(0, rows * nw, body, 0)


@functools.partial(
    pl.kernel,
    out_type=jax.ShapeDtypeStruct((_NC, _NP, _DEGW), jnp.float32),
    mesh=_mesh,
    compiler_params=pltpu.CompilerParams(use_tc_tiling_on_sc=False),
    scratch_types=[
        pltpu.VMEM((_CHUNK, _DEGW), jnp.float32),
        pltpu.VMEM((_RPT, _DEGW), jnp.float32),
        pltpu.VMEM((_CHUNK,), jnp.int32),
        pltpu.VMEM_SHARED((_NP, _DEGW), jnp.float32),
    ],
)
def _deg_scatter(dst_hbm, out_hbm, ones_v, stage, idx_d, acc):
  c = lax.axis_index("c")
  s = lax.axis_index("s")
  wid = s * _NC + c

  ov = jnp.ones((16,), jnp.float32)

  def fill_ones(i, carry):
    r = i // (_DEGW // 16)
    cs = (i % (_DEGW // 16)) * 16
    ones_v[r, pl.ds(cs, 16)] = ov
    return carry

  lax.fori_loop(0, _CHUNK * (_DEGW // 16), fill_ones, 0)
  _zero_fill(stage, _RPT, _DEGW)
  pltpu.sync_copy(stage, acc.at[pl.ds(s * _RPT, _RPT)])
  plsc.subcore_barrier()

  nk = _NCHUNK // _NW + jnp.where(wid < _NCHUNK % _NW, 1, 0)

  def body(k, carry):
    base = (wid + k * _NW) * _CHUNK
    pltpu.sync_copy(dst_hbm.at[pl.ds(base, _CHUNK)], idx_d)
    pltpu.sync_copy(ones_v, acc.at[idx_d], add=True)
    return carry

  lax.fori_loop(0, nk, body, 0)
  plsc.subcore_barrier()
  pltpu.sync_copy(
      acc.at[pl.ds(s * _RPT, _RPT)], out_hbm.at[c, pl.ds(s * _RPT, _RPT)]
  )


@functools.partial(
    pl.kernel,
    out_type=jax.ShapeDtypeStruct((_NC, _NP, _H), jnp.float32),
    mesh=_mesh,
    compiler_params=pltpu.CompilerParams(use_tc_tiling_on_sc=False),
    scratch_types=[
        pltpu.VMEM((_CHUNK,), jnp.int32),
        pltpu.VMEM((_CHUNK,), jnp.int32),
        pltpu.VMEM((_CHUNK, _H), jnp.float32),
        pltpu.VMEM((_RPT, _H), jnp.float32),
        pltpu.VMEM_SHARED((_NP, _H), jnp.float32),
        pltpu.SemaphoreType.DMA,
    ],
)
def _conv_scatter(src_hbm, dst_hbm, hws_hbm, out_hbm, idx_s, idx_d, rows, stage, acc, sem):
  c = lax.axis_index("c")
  s = lax.axis_index("s")
  wid = s * _NC + c

  _zero_fill(stage, _RPT, _H)
  pltpu.sync_copy(stage, acc.at[pl.ds(s * _RPT, _RPT)])
  plsc.subcore_barrier()

  nk = _NCHUNK // _NW + jnp.where(wid < _NCHUNK % _NW, 1, 0)

  def body(k, carry):
    base = (wid + k * _NW) * _CHUNK
    pltpu.sync_copy(src_hbm.at[pl.ds(base, _CHUNK)], idx_s)
    pltpu.sync_copy(dst_hbm.at[pl.ds(base, _CHUNK)], idx_d)
    pltpu.async_copy(hws_hbm.at[idx_s], rows, sem).wait()
    pltpu.sync_copy(rows, acc.at[idx_d], add=True)
    return carry

  lax.fori_loop(0, nk, body, 0)
  plsc.subcore_barrier()
  pltpu.sync_copy(
      acc.at[pl.ds(s * _RPT, _RPT)], out_hbm.at[c, pl.ds(s * _RPT, _RPT)]
  )


def _bn_stats(h):
  m = jnp.mean(h, axis=0, keepdims=True)
  v = jnp.mean((h - m) ** 2, axis=0, keepdims=True)
  return m, v


def _tc_prep(deg_ref, x_ref, w1_ref, dinv_ref, hws1_ref):
  deg = deg_ref[0, 0:_N, 0:1] + deg_ref[1, 0:_N, 0:1] + 1.0
  dinv = lax.rsqrt(deg)
  dinv_ref[...] = dinv
  hw = jnp.dot(x_ref[...], w1_ref[...], preferred_element_type=jnp.float32)
  hws1_ref[...] = hw * dinv


def _tc_mid1(acc_ref, hws_ref, dinv_ref, b_ref, g_ref, be_ref, w2_ref,
             h1_ref, hws2_ref):
  dinv = dinv_ref[...]
  agg = (acc_ref[0, 0:_N] + acc_ref[1, 0:_N] + hws_ref[...]) * dinv + b_ref[...]
  m, v = _bn_stats(agg)
  h1 = jnp.maximum((agg - m) * lax.rsqrt(v + 1e-5) * g_ref[...] + be_ref[...], 0.0)
  h1_ref[...] = h1
  hws2_ref[...] = (
      jnp.dot(h1, w2_ref[...], preferred_element_type=jnp.float32) * dinv
  )


def _tc_mid2(acc_ref, hws_ref, dinv_ref, b_ref, g_ref, be_ref, h1_ref, w3_ref,
             hws3_ref):
  dinv = dinv_ref[...]
  agg = (acc_ref[0, 0:_N] + acc_ref[1, 0:_N] + hws_ref[...]) * dinv + b_ref[...]
  m, v = _bn_stats(agg)
  bn2 = (agg - m) * lax.rsqrt(v + 1e-5) * g_ref[...] + be_ref[...]
  h2 = jnp.maximum(bn2 + h1_ref[...], 0.0)
  hws3_ref[...] = (
      jnp.dot(h2, w3_ref[...], preferred_element_type=jnp.float32) * dinv
  )


def _tc_final(acc_ref, hws_ref, dinv_ref, b_ref, g_ref, be_ref, batch_ref,
              sigma_ref, wf1_ref, bf1_ref, wf2_ref, bf2_ref, wfc_ref, bfc_ref,
              out_ref):
  dinv = dinv_ref[...]
  agg = (acc_ref[0, 0:_N] + acc_ref[1, 0:_N] + hws_ref[...]) * dinv + b_ref[...]
  m, v = _bn_stats(agg)
  h3 = jnp.maximum((agg - m) * lax.rsqrt(v + 1e-5) * g_ref[...] + be_ref[...], 0.0)

  gids = lax.broadcasted_iota(jnp.int32, (_G, 1), 0)
  onehot = (batch_ref[...] == gids).astype(jnp.float32)
  sums = jnp.dot(onehot, h3, preferred_element_type=jnp.float32)
  cnt = jnp.sum(onehot, axis=1, keepdims=True)
  gemb = sums / jnp.maximum(cnt, 1.0)

  f = jnp.maximum(
      jnp.dot(sigma_ref[...], wf1_ref[...], preferred_element_type=jnp.float32)
      + bf1_ref[...], 0.0)
  f = jnp.maximum(
      jnp.dot(f, wf2_ref[...], preferred_element_type=jnp.float32)
      + bf2_ref[...], 0.0)

  out_ref[...] = (
      jnp.dot(gemb, wfc_ref[0:_H, :], preferred_element_type=jnp.float32)
      + jnp.dot(f, wfc_ref[_H:, :], preferred_element_type=jnp.float32)
      + bfc_ref[...]
  )


def kernel(x, edge_index, batch, sigma, W1, b1, W2, b2, W3, b3, g1, be1, g2,
           be2, g3, be3, Wf1, bf1, Wf2, bf2, Wfc, bfc):
  src = edge_index[0]
  dst = edge_index[1]
  f32 = jnp.float32

  degp = _deg_scatter(dst)

  dinv, hws1 = pl.pallas_call(
      _tc_prep,
      out_shape=[
          jax.ShapeDtypeStruct((_N, 1), f32),
          jax.ShapeDtypeStruct((_N, _H), f32),
      ],
  )(degp, x, W1)

  acc1 = _conv_scatter(src, dst, hws1)

  h1, hws2 = pl.pallas_call(
      _tc_mid1,
      out_shape=[
          jax.ShapeDtypeStruct((_N, _H), f32),
          jax.ShapeDtypeStruct((_N, _H), f32),
      ],
  )(acc1, hws1, dinv, b1.reshape(1, _H), g1.reshape(1, _H),
    be1.reshape(1, _H), W2)

  acc2 = _conv_scatter(src, dst, hws2)

  hws3, = pl.pallas_call(
      _tc_mid2,
      out_shape=[jax.ShapeDtypeStruct((_N, _H), f32)],
  )(acc2, hws2, dinv, b2.reshape(1, _H), g2.reshape(1, _H),
    be2.reshape(1, _H), h1, W3)

  acc3 = _conv_scatter(src, dst, hws3)

  out2d = pl.pallas_call(
      _tc_final,
      out_shape=jax.ShapeDtypeStruct((_G, 1), f32),
  )(acc3, hws3, dinv, b3.reshape(1, _H), g3.reshape(1, _H),
    be3.reshape(1, _H), batch.reshape(1, _N), sigma, Wf1,
    bf1.reshape(1, 2 * _H), Wf2, bf2.reshape(1, _H), Wfc,
    bfc.reshape(1, 1))

  return out2d.reshape(_G)


# speedup vs baseline: 33.5468x; 2.1235x over previous
import functools

import jax
import jax.numpy as jnp
from jax import lax
from jax.experimental import pallas as pl
from jax.experimental.pallas import tpu as pltpu
from jax.experimental.pallas import tpu_sc as plsc

_N = 10000
_E = 320000
_D = 128
_H = 64
_G = 16

_NC = 2
_NS = 16
_NW = _NC * _NS
_CHUNK = 128
_NCHUNK = _E // _CHUNK
_RPT = 632
_NP = _RPT * _NS
_DEGW = 16
_NKW = 78
_ZR = 79

_mesh = plsc.VectorSubcoreMesh(
    core_axis_name="c", subcore_axis_name="s", num_cores=_NC, num_subcores=_NS
)


def _zero_fill(ref, rows, width):
  zv = jnp.zeros((16,), jnp.float32)
  nw = width // 16

  def body(i, carry):
    r = i // nw
    cs = (i % nw) * 16
    ref[r, pl.ds(cs, 16)] = zv
    return carry

  lax.fori_loop(0, rows * nw, body, 0)


@functools.partial(
    pl.kernel,
    out_type=jax.ShapeDtypeStruct((_NC, _NP, _DEGW), jnp.float32),
    mesh=_mesh,
    compiler_params=pltpu.CompilerParams(use_tc_tiling_on_sc=False),
    scratch_types=[
        pltpu.VMEM((_CHUNK, _DEGW), jnp.float32),
        pltpu.VMEM((_RPT, _DEGW), jnp.float32),
        pltpu.VMEM((_CHUNK,), jnp.int32),
        pltpu.VMEM_SHARED((_NP, _DEGW), jnp.float32),
    ],
)
def _deg_scatter(dst_hbm, out_hbm, ones_v, stage, idx_d, acc):
  c = lax.axis_index("c")
  s = lax.axis_index("s")
  wid = s * _NC + c

  ov = jnp.ones((16,), jnp.float32)

  def fill_ones(i, carry):
    r = i // (_DEGW // 16)
    cs = (i % (_DEGW // 16)) * 16
    ones_v[r, pl.ds(cs, 16)] = ov
    return carry

  lax.fori_loop(0, _CHUNK * (_DEGW // 16), fill_ones, 0)
  _zero_fill(stage, _RPT, _DEGW)
  pltpu.sync_copy(stage, acc.at[pl.ds(s * _RPT, _RPT)])
  plsc.subcore_barrier()

  nk = _NCHUNK // _NW + jnp.where(wid < _NCHUNK % _NW, 1, 0)

  def body(k, carry):
    base = (wid + k * _NW) * _CHUNK
    pltpu.sync_copy(dst_hbm.at[pl.ds(base, _CHUNK)], idx_d)
    pltpu.sync_copy(ones_v, acc.at[idx_d], add=True)
    return carry

  lax.fori_loop(0, nk, body, 0)
  plsc.subcore_barrier()
  pltpu.sync_copy(
      acc.at[pl.ds(s * _RPT, _RPT)], out_hbm.at[c, pl.ds(s * _RPT, _RPT)]
  )


@functools.partial(
    pl.kernel,
    out_type=jax.ShapeDtypeStruct((_NC, _NP, _H), jnp.float32),
    mesh=_mesh,
    compiler_params=pltpu.CompilerParams(use_tc_tiling_on_sc=False),
    scratch_types=[
        [pltpu.VMEM((2, _CHUNK), jnp.int32) for _ in range(4)],
        [pltpu.VMEM((_CHUNK, _H), jnp.float32) for _ in range(4)],
        pltpu.VMEM((_ZR, _H), jnp.float32),
        pltpu.VMEM_SHARED((_NP, _H), jnp.float32),
        [pltpu.SemaphoreType.DMA for _ in range(4)],
        [pltpu.SemaphoreType.DMA for _ in range(4)],
        [pltpu.SemaphoreType.DMA for _ in range(4)],
    ],
)
def _conv_scatter(ei_hbm, hws_hbm, out_hbm, idx, rows, stage, acc, isem, gsem,
                  ssem):
  c = lax.axis_index("c")
  s = lax.axis_index("s")
  wid = s * _NC + c

  _zero_fill(stage, _ZR, _H)
  for i in range(_RPT // _ZR):
    pltpu.sync_copy(stage, acc.at[pl.ds(s * _RPT + i * _ZR, _ZR)])
  plsc.subcore_barrier()

  def fetch_idx(k, j):
    base = (wid + k * _NW) * _CHUNK
    pltpu.async_copy(ei_hbm.at[:, pl.ds(base, _CHUNK)], idx[j], isem[j])

  def wait_idx(j):
    pltpu.make_async_copy(ei_hbm.at[:, pl.ds(0, _CHUNK)], idx[j], isem[j]).wait()

  def gather(j):
    pltpu.async_copy(hws_hbm.at[idx[j].at[0]], rows[j], gsem[j])

  def wait_gather(j):
    pltpu.make_async_copy(hws_hbm.at[idx[j].at[0]], rows[j], gsem[j]).wait()

  def scatter(j):
    pltpu.async_copy(rows[j], acc.at[idx[j].at[1]], ssem[j], add=True)

  def wait_scatter(j):
    pltpu.make_async_copy(rows[j], acc.at[idx[j].at[1]], ssem[j]).wait()

  fetch_idx(0, 0)
  fetch_idx(1, 1)
  wait_idx(0); gather(0); fetch_idx(2, 2)
  wait_idx(1); gather(1); wait_gather(0); scatter(0); fetch_idx(3, 3)
  wait_idx(2); gather(2); wait_gather(1); scatter(1); wait_scatter(0); fetch_idx(4, 0)
  wait_idx(3); gather(3); wait_gather(2); scatter(2); wait_scatter(1); fetch_idx(5, 1)

  def piped(kk, carry):
    for j in range(4):
      k = kk * 4 + j
      wait_idx(j)
      gather(j)
      wait_gather((j + 3) % 4)
      scatter((j + 3) % 4)
      wait_scatter((j + 2) % 4)
      fetch_idx(k + 2, (j + 2) % 4)
    return carry

  lax.fori_loop(1, _NKW // 4, piped, 0)

  wait_idx(0); gather(0); wait_gather(3); scatter(3); wait_scatter(2)
  wait_idx(1); gather(1); wait_gather(0); scatter(0); wait_scatter(3)
  wait_gather(1); scatter(1); wait_scatter(0); wait_scatter(1)

  @pl.when(wid < _NCHUNK - _NKW * _NW)
  def _():
    base = (_NKW * _NW + wid) * _CHUNK
    pltpu.sync_copy(ei_hbm.at[:, pl.ds(base, _CHUNK)], idx[2])
    pltpu.async_copy(hws_hbm.at[idx[2].at[0]], rows[2], gsem[2]).wait()
    pltpu.async_copy(rows[2], acc.at[idx[2].at[1]], ssem[2], add=True).wait()

  plsc.subcore_barrier()
  pltpu.sync_copy(
      acc.at[pl.ds(s * _RPT, _RPT)], out_hbm.at[c, pl.ds(s * _RPT, _RPT)]
  )


def _bn_stats(h):
  m = jnp.mean(h, axis=0, keepdims=True)
  v = jnp.mean((h - m) ** 2, axis=0, keepdims=True)
  return m, v


def _tc_mm1(x_ref, w1_ref, hw1_ref):
  hw1_ref[...] = jnp.dot(x_ref[...], w1_ref[...],
                         preferred_element_type=jnp.float32)


def _tc_scale(deg_ref, hw_ref, dinv_ref, hws1_ref):
  deg = deg_ref[0, 0:_N, 0:1] + deg_ref[1, 0:_N, 0:1] + 1.0
  dinv = lax.rsqrt(deg)
  dinv_ref[...] = dinv
  hws1_ref[...] = hw_ref[...] * dinv


def _tc_mid1(acc_ref, hws_ref, dinv_ref, b_ref, g_ref, be_ref, w2_ref,
             h1_ref, hws2_ref):
  dinv = dinv_ref[...]
  agg = (acc_ref[0, 0:_N] + acc_ref[1, 0:_N] + hws_ref[...]) * dinv + b_ref[...]
  m, v = _bn_stats(agg)
  h1 = jnp.maximum((agg - m) * lax.rsqrt(v + 1e-5) * g_ref[...] + be_ref[...], 0.0)
  h1_ref[...] = h1
  hws2_ref[...] = (
      jnp.dot(h1, w2_ref[...], preferred_element_type=jnp.float32) * dinv
  )


def _tc_mid2(acc_ref, hws_ref, dinv_ref, b_ref, g_ref, be_ref, h1_ref, w3_ref,
             hws3_ref):
  dinv = dinv_ref[...]
  agg = (acc_ref[0, 0:_N] + acc_ref[1, 0:_N] + hws_ref[...]) * dinv + b_ref[...]
  m, v = _bn_stats(agg)
  bn2 = (agg - m) * lax.rsqrt(v + 1e-5) * g_ref[...] + be_ref[...]
  h2 = jnp.maximum(bn2 + h1_ref[...], 0.0)
  hws3_ref[...] = (
      jnp.dot(h2, w3_ref[...], preferred_element_type=jnp.float32) * dinv
  )


def _tc_final(acc_ref, hws_ref, dinv_ref, b_ref, g_ref, be_ref, batch_ref,
              sigma_ref, wf1_ref, bf1_ref, wf2_ref, bf2_ref, wfc_ref, bfc_ref,
              out_ref):
  dinv = dinv_ref[...]
  agg = (acc_ref[0, 0:_N] + acc_ref[1, 0:_N] + hws_ref[...]) * dinv + b_ref[...]
  m, v = _bn_stats(agg)
  h3 = jnp.maximum((agg - m) * lax.rsqrt(v + 1e-5) * g_ref[...] + be_ref[...], 0.0)

  gids = lax.broadcasted_iota(jnp.int32, (_G, 1), 0)
  onehot = (batch_ref[...] == gids).astype(jnp.float32)
  sums = jnp.dot(onehot, h3, preferred_element_type=jnp.float32)
  cnt = jnp.sum(onehot, axis=1, keepdims=True)
  gemb = sums / jnp.maximum(cnt, 1.0)

  f = jnp.maximum(
      jnp.dot(sigma_ref[...], wf1_ref[...], preferred_element_type=jnp.float32)
      + bf1_ref[...], 0.0)
  f = jnp.maximum(
      jnp.dot(f, wf2_ref[...], preferred_element_type=jnp.float32)
      + bf2_ref[...], 0.0)

  out_ref[...] = (
      jnp.dot(gemb, wfc_ref[0:_H, :], preferred_element_type=jnp.float32)
      + jnp.dot(f, wfc_ref[_H:, :], preferred_element_type=jnp.float32)
      + bfc_ref[...]
  )


def kernel(x, edge_index, batch, sigma, W1, b1, W2, b2, W3, b3, g1, be1, g2,
           be2, g3, be3, Wf1, bf1, Wf2, bf2, Wfc, bfc):
  src = edge_index[0]
  dst = edge_index[1]
  f32 = jnp.float32

  degp = _deg_scatter(dst)

  hw1 = pl.pallas_call(
      _tc_mm1,
      out_shape=jax.ShapeDtypeStruct((_N, _H), f32),
  )(x, W1)

  dinv, hws1 = pl.pallas_call(
      _tc_scale,
      out_shape=[
          jax.ShapeDtypeStruct((_N, 1), f32),
          jax.ShapeDtypeStruct((_N, _H), f32),
      ],
  )(degp, hw1)

  acc1 = _conv_scatter(edge_index, hws1)

  h1, hws2 = pl.pallas_call(
      _tc_mid1,
      out_shape=[
          jax.ShapeDtypeStruct((_N, _H), f32),
          jax.ShapeDtypeStruct((_N, _H), f32),
      ],
  )(acc1, hws1, dinv, b1.reshape(1, _H), g1.reshape(1, _H),
    be1.reshape(1, _H), W2)

  acc2 = _conv_scatter(edge_index, hws2)

  hws3, = pl.pallas_call(
      _tc_mid2,
      out_shape=[jax.ShapeDtypeStruct((_N, _H), f32)],
  )(acc2, hws2, dinv, b2.reshape(1, _H), g2.reshape(1, _H),
    be2.reshape(1, _H), h1, W3)

  acc3 = _conv_scatter(edge_index, hws3)

  out2d = pl.pallas_call(
      _tc_final,
      out_shape=jax.ShapeDtypeStruct((_G, 1), f32),
  )(acc3, hws3, dinv, b3.reshape(1, _H), g3.reshape(1, _H),
    be3.reshape(1, _H), batch.reshape(1, _N), sigma, Wf1,
    bf1.reshape(1, 2 * _H), Wf2, bf2.reshape(1, _H), Wfc,
    bfc.reshape(1, 1))

  return out2d.reshape(_G)


# speedup vs baseline: 39.4702x; 1.1766x over previous
import functools

import jax
import jax.numpy as jnp
from jax import lax
from jax.experimental import pallas as pl
from jax.experimental.pallas import tpu as pltpu
from jax.experimental.pallas import tpu_sc as plsc

_N = 10000
_E = 320000
_D = 128
_H = 64
_G = 16

_NC = 2
_NS = 16
_NW = _NC * _NS
_CHUNK = 128
_NCHUNK = _E // _CHUNK
_RPT = 632
_NP = _RPT * _NS
_DEGW = 16
_NSS = 39
_ZR = 79

_mesh = plsc.VectorSubcoreMesh(
    core_axis_name="c", subcore_axis_name="s", num_cores=_NC, num_subcores=_NS
)


def _zero_fill(ref, rows, width):
  zv = jnp.zeros((16,), jnp.float32)
  nw = width // 16

  def body(i, carry):
    r = i // nw
    cs = (i % nw) * 16
    ref[r, pl.ds(cs, 16)] = zv
    return carry

  lax.fori_loop(0, rows * nw, body, 0)


@functools.partial(
    pl.kernel,
    out_type=jax.ShapeDtypeStruct((_NC, _NP, _DEGW), jnp.float32),
    mesh=_mesh,
    compiler_params=pltpu.CompilerParams(use_tc_tiling_on_sc=False),
    scratch_types=[
        pltpu.VMEM((_CHUNK, _DEGW), jnp.float32),
        pltpu.VMEM((_ZR, _DEGW), jnp.float32),
        [pltpu.VMEM((2, _CHUNK), jnp.int32) for _ in range(3)],
        pltpu.VMEM_SHARED((_NP, _DEGW), jnp.float32),
        [pltpu.SemaphoreType.DMA for _ in range(3)],
        [pltpu.SemaphoreType.DMA for _ in range(3)],
    ],
)
def _deg_scatter(ei_hbm, out_hbm, ones_v, stage, idx, acc, isem, ssem):
  c = lax.axis_index("c")
  s = lax.axis_index("s")
  wid = s * _NC + c

  ov = jnp.ones((16,), jnp.float32)

  def fill_ones(i, carry):
    r = i // (_DEGW // 16)
    cs = (i % (_DEGW // 16)) * 16
    ones_v[r, pl.ds(cs, 16)] = ov
    return carry

  lax.fori_loop(0, _CHUNK * (_DEGW // 16), fill_ones, 0)
  _zero_fill(stage, _ZR, _DEGW)
  for i in range(_RPT // _ZR):
    pltpu.sync_copy(stage, acc.at[pl.ds(s * _RPT + i * _ZR, _ZR)])
  plsc.subcore_barrier()

  def fetch_idx(t, j):
    p = wid + t * _NW
    pltpu.async_copy(ei_hbm.at[1, pl.ds(2 * p, 2), :], idx[j], isem[j])

  def wait_idx(j):
    pltpu.make_async_copy(ei_hbm.at[1, pl.ds(0, 2), :], idx[j], isem[j]).wait()

  def scatter(j):
    for u in range(2):
      pltpu.async_copy(ones_v, acc.at[idx[j].at[u]], ssem[j], add=True)

  def wait_scatter(j):
    for u in range(2):
      pltpu.make_async_copy(ones_v, acc.at[idx[j].at[u]], ssem[j]).wait()

  fetch_idx(0, 0)
  fetch_idx(1, 1)
  wait_idx(0); scatter(0); fetch_idx(2, 2)
  wait_idx(1); scatter(1)

  def piped(i, carry):
    for u in range(3):
      t = 2 + i * 3 + u
      j = (2 + u) % 3
      jp = (j + 1) % 3
      wait_scatter(jp)
      fetch_idx(t + 1, jp)
      wait_idx(j)
      scatter(j)
    return carry

  lax.fori_loop(0, (_NSS - 3) // 3, piped, 0)

  _jl = (_NSS - 1) % 3
  wait_scatter((_jl + 1) % 3)
  wait_idx(_jl); scatter(_jl)
  wait_scatter((_jl + 2) % 3); wait_scatter(_jl)

  @pl.when(wid < _NCHUNK - 2 * _NSS * _NW)
  def _():
    cid = 2 * _NSS * _NW + wid
    pltpu.sync_copy(ei_hbm.at[1, pl.ds(cid, 1), :], idx[0].at[pl.ds(0, 1), :])
    pltpu.async_copy(ones_v, acc.at[idx[0].at[0]], ssem[0], add=True).wait()

  plsc.subcore_barrier()
  pltpu.sync_copy(
      acc.at[pl.ds(s * _RPT, _RPT)], out_hbm.at[c, pl.ds(s * _RPT, _RPT)]
  )


@functools.partial(
    pl.kernel,
    out_type=jax.ShapeDtypeStruct((_NC, _NP, _H), jnp.float32),
    mesh=_mesh,
    compiler_params=pltpu.CompilerParams(use_tc_tiling_on_sc=False),
    scratch_types=[
        [pltpu.VMEM((2, 2, _CHUNK), jnp.int32) for _ in range(3)],
        [pltpu.VMEM((2 * _CHUNK, _H), jnp.float32) for _ in range(3)],
        pltpu.VMEM((_ZR, _H), jnp.float32),
        pltpu.VMEM_SHARED((_NP, _H), jnp.float32),
        [pltpu.SemaphoreType.DMA for _ in range(3)],
        [pltpu.SemaphoreType.DMA for _ in range(3)],
        [pltpu.SemaphoreType.DMA for _ in range(3)],
    ],
)
def _conv_scatter(ei_hbm, hws_hbm, out_hbm, idx, rows, stage, acc, isem, gsem,
                  ssem):
  c = lax.axis_index("c")
  s = lax.axis_index("s")
  wid = s * _NC + c

  _zero_fill(stage, _ZR, _H)
  for i in range(_RPT // _ZR):
    pltpu.sync_copy(stage, acc.at[pl.ds(s * _RPT + i * _ZR, _ZR)])
  plsc.subcore_barrier()

  def fetch_idx(t, j):
    p = wid + t * _NW
    pltpu.async_copy(ei_hbm.at[:, pl.ds(2 * p, 2), :], idx[j], isem[j])

  def wait_idx(j):
    pltpu.make_async_copy(
        ei_hbm.at[:, pl.ds(0, 2), :], idx[j], isem[j]).wait()

  def gather(j):
    for u in range(2):
      pltpu.async_copy(hws_hbm.at[idx[j].at[0, u]],
                       rows[j].at[pl.ds(u * _CHUNK, _CHUNK)], gsem[j])

  def wait_gather(j):
    for u in range(2):
      pltpu.make_async_copy(hws_hbm.at[idx[j].at[0, u]],
                            rows[j].at[pl.ds(u * _CHUNK, _CHUNK)],
                            gsem[j]).wait()

  def scatter(j):
    for u in range(2):
      pltpu.async_copy(rows[j].at[pl.ds(u * _CHUNK, _CHUNK)],
                       acc.at[idx[j].at[1, u]], ssem[j], add=True)

  def wait_scatter(j):
    for u in range(2):
      pltpu.make_async_copy(rows[j].at[pl.ds(u * _CHUNK, _CHUNK)],
                            acc.at[idx[j].at[1, u]], ssem[j]).wait()

  fetch_idx(0, 0)
  fetch_idx(1, 1)
  wait_idx(0); gather(0); fetch_idx(2, 2)
  wait_idx(1); gather(1); wait_gather(0); scatter(0)

  def piped(i, carry):
    for u in range(3):
      t = 2 + i * 3 + u
      j = (2 + u) % 3
      jp = (j + 1) % 3
      jm = (j + 2) % 3
      wait_scatter(jp)
      fetch_idx(t + 1, jp)
      wait_idx(j)
      gather(j)
      wait_gather(jm)
      scatter(jm)
    return carry

  lax.fori_loop(0, (_NSS - 3) // 3, piped, 0)

  _jl = (_NSS - 1) % 3
  wait_scatter((_jl + 1) % 3)
  wait_idx(_jl); gather(_jl)
  wait_gather((_jl + 2) % 3); scatter((_jl + 2) % 3)
  wait_gather(_jl); scatter(_jl)
  wait_scatter((_jl + 2) % 3); wait_scatter(_jl)

  @pl.when(wid < _NCHUNK - 2 * _NSS * _NW)
  def _():
    cid = 2 * _NSS * _NW + wid
    pltpu.sync_copy(ei_hbm.at[:, pl.ds(cid, 1), :], idx[0].at[:, pl.ds(0, 1), :])
    pltpu.async_copy(hws_hbm.at[idx[0].at[0, 0]],
                     rows[0].at[pl.ds(0, _CHUNK)], gsem[0]).wait()
    pltpu.async_copy(rows[0].at[pl.ds(0, _CHUNK)],
                     acc.at[idx[0].at[1, 0]], ssem[0], add=True).wait()

  plsc.subcore_barrier()
  pltpu.sync_copy(
      acc.at[pl.ds(s * _RPT, _RPT)], out_hbm.at[c, pl.ds(s * _RPT, _RPT)]
  )


def _bn_stats(h):
  m = jnp.mean(h, axis=0, keepdims=True)
  v = jnp.mean((h - m) ** 2, axis=0, keepdims=True)
  return m, v


def _tc_mm1(x_ref, w1_ref, hw1_ref):
  hw1_ref[...] = jnp.dot(x_ref[...], w1_ref[...],
                         preferred_element_type=jnp.float32)


def _tc_scale(deg_ref, hw_ref, dinv_ref, hws1_ref):
  deg = deg_ref[0, 0:_N, 0:1] + deg_ref[1, 0:_N, 0:1] + 1.0
  dinv = lax.rsqrt(deg)
  dinv_ref[...] = dinv
  hws1_ref[...] = hw_ref[...] * dinv


def _tc_mid1(acc_ref, hws_ref, dinv_ref, b_ref, g_ref, be_ref, w2_ref,
             h1_ref, hws2_ref):
  dinv = dinv_ref[...]
  agg = (acc_ref[0, 0:_N] + acc_ref[1, 0:_N] + hws_ref[...]) * dinv + b_ref[...]
  m, v = _bn_stats(agg)
  h1 = jnp.maximum((agg - m) * lax.rsqrt(v + 1e-5) * g_ref[...] + be_ref[...], 0.0)
  h1_ref[...] = h1
  hws2_ref[...] = (
      jnp.dot(h1, w2_ref[...], preferred_element_type=jnp.float32) * dinv
  )


def _tc_mid2(acc_ref, hws_ref, dinv_ref, b_ref, g_ref, be_ref, h1_ref, w3_ref,
             hws3_ref):
  dinv = dinv_ref[...]
  agg = (acc_ref[0, 0:_N] + acc_ref[1, 0:_N] + hws_ref[...]) * dinv + b_ref[...]
  m, v = _bn_stats(agg)
  bn2 = (agg - m) * lax.rsqrt(v + 1e-5) * g_ref[...] + be_ref[...]
  h2 = jnp.maximum(bn2 + h1_ref[...], 0.0)
  hws3_ref[...] = (
      jnp.dot(h2, w3_ref[...], preferred_element_type=jnp.float32) * dinv
  )


def _tc_final(acc_ref, hws_ref, dinv_ref, b_ref, g_ref, be_ref, batch_ref,
              sigma_ref, wf1_ref, bf1_ref, wf2_ref, bf2_ref, wfc_ref, bfc_ref,
              out_ref):
  dinv = dinv_ref[...]
  agg = (acc_ref[0, 0:_N] + acc_ref[1, 0:_N] + hws_ref[...]) * dinv + b_ref[...]
  m, v = _bn_stats(agg)
  h3 = jnp.maximum((agg - m) * lax.rsqrt(v + 1e-5) * g_ref[...] + be_ref[...], 0.0)

  gids = lax.broadcasted_iota(jnp.int32, (_G, 1), 0)
  onehot = (batch_ref[...] == gids).astype(jnp.float32)
  sums = jnp.dot(onehot, h3, preferred_element_type=jnp.float32)
  cnt = jnp.sum(onehot, axis=1, keepdims=True)
  gemb = sums / jnp.maximum(cnt, 1.0)

  f = jnp.maximum(
      jnp.dot(sigma_ref[...], wf1_ref[...], preferred_element_type=jnp.float32)
      + bf1_ref[...], 0.0)
  f = jnp.maximum(
      jnp.dot(f, wf2_ref[...], preferred_element_type=jnp.float32)
      + bf2_ref[...], 0.0)

  out_ref[...] = (
      jnp.dot(gemb, wfc_ref[0:_H, :], preferred_element_type=jnp.float32)
      + jnp.dot(f, wfc_ref[_H:, :], preferred_element_type=jnp.float32)
      + bfc_ref[...]
  )


def kernel(x, edge_index, batch, sigma, W1, b1, W2, b2, W3, b3, g1, be1, g2,
           be2, g3, be3, Wf1, bf1, Wf2, bf2, Wfc, bfc):
  f32 = jnp.float32
  ei3 = edge_index.reshape(2, _NCHUNK, _CHUNK)

  degp = _deg_scatter(ei3)

  hw1 = pl.pallas_call(
      _tc_mm1,
      out_shape=jax.ShapeDtypeStruct((_N, _H), f32),
  )(x, W1)

  dinv, hws1 = pl.pallas_call(
      _tc_scale,
      out_shape=[
          jax.ShapeDtypeStruct((_N, 1), f32),
          jax.ShapeDtypeStruct((_N, _H), f32),
      ],
  )(degp, hw1)

  acc1 = _conv_scatter(ei3, hws1)

  h1, hws2 = pl.pallas_call(
      _tc_mid1,
      out_shape=[
          jax.ShapeDtypeStruct((_N, _H), f32),
          jax.ShapeDtypeStruct((_N, _H), f32),
      ],
  )(acc1, hws1, dinv, b1.reshape(1, _H), g1.reshape(1, _H),
    be1.reshape(1, _H), W2)

  acc2 = _conv_scatter(ei3, hws2)

  hws3, = pl.pallas_call(
      _tc_mid2,
      out_shape=[jax.ShapeDtypeStruct((_N, _H), f32)],
  )(acc2, hws2, dinv, b2.reshape(1, _H), g2.reshape(1, _H),
    be2.reshape(1, _H), h1, W3)

  acc3 = _conv_scatter(ei3, hws3)

  out2d = pl.pallas_call(
      _tc_final,
      out_shape=jax.ShapeDtypeStruct((_G, 1), f32),
  )(acc3, hws3, dinv, b3.reshape(1, _H), g3.reshape(1, _H),
    be3.reshape(1, _H), batch.reshape(1, _N), sigma, Wf1,
    bf1.reshape(1, 2 * _H), Wf2, bf2.reshape(1, _H), Wfc,
    bfc.reshape(1, 1))

  return out2d.reshape(_G)


# speedup vs baseline: 39.5558x; 1.0022x over previous
import functools

import jax
import jax.numpy as jnp
from jax import lax
from jax.experimental import pallas as pl
from jax.experimental.pallas import tpu as pltpu
from jax.experimental.pallas import tpu_sc as plsc

_N = 10000
_E = 320000
_D = 128
_H = 64
_G = 16

_NC = 2
_NS = 16
_NW = _NC * _NS
_CHUNK = 128
_NCHUNK = _E // _CHUNK
_RPT = 632
_NP = _RPT * _NS
_DEGW = 16
_NSS = 39
_ZR = 79

_mesh = plsc.VectorSubcoreMesh(
    core_axis_name="c", subcore_axis_name="s", num_cores=_NC, num_subcores=_NS
)


def _zero_fill(ref, rows, width):
  zv = jnp.zeros((16,), jnp.float32)
  nw = width // 16

  def body(i, carry):
    r = i // nw
    cs = (i % nw) * 16
    ref[r, pl.ds(cs, 16)] = zv
    return carry

  lax.fori_loop(0, rows * nw, body, 0)


@functools.partial(
    pl.kernel,
    out_type=jax.ShapeDtypeStruct((_NC, _NP, _DEGW), jnp.float32),
    mesh=_mesh,
    compiler_params=pltpu.CompilerParams(use_tc_tiling_on_sc=False),
    scratch_types=[
        pltpu.VMEM((_CHUNK, _DEGW), jnp.float32),
        pltpu.VMEM((_ZR, _DEGW), jnp.float32),
        [pltpu.VMEM((2, _CHUNK), jnp.int32) for _ in range(3)],
        pltpu.VMEM_SHARED((_NP, _DEGW), jnp.float32),
        [pltpu.SemaphoreType.DMA for _ in range(3)],
        [pltpu.SemaphoreType.DMA for _ in range(3)],
    ],
)
def _deg_scatter(ei_hbm, out_hbm, ones_v, stage, idx, acc, isem, ssem):
  c = lax.axis_index("c")
  s = lax.axis_index("s")
  wid = s * _NC + c

  ov = jnp.ones((16,), jnp.float32)

  def fill_ones(i, carry):
    r = i // (_DEGW // 16)
    cs = (i % (_DEGW // 16)) * 16
    ones_v[r, pl.ds(cs, 16)] = ov
    return carry

  lax.fori_loop(0, _CHUNK * (_DEGW // 16), fill_ones, 0)
  _zero_fill(stage, _ZR, _DEGW)
  for i in range(_RPT // _ZR):
    pltpu.sync_copy(stage, acc.at[pl.ds(s * _RPT + i * _ZR, _ZR)])
  plsc.subcore_barrier()

  def fetch_idx(t, j):
    p = wid + t * _NW
    pltpu.async_copy(ei_hbm.at[1, pl.ds(2 * p, 2), :], idx[j], isem[j])

  def wait_idx(j):
    pltpu.make_async_copy(ei_hbm.at[1, pl.ds(0, 2), :], idx[j], isem[j]).wait()

  def scatter(j):
    for u in range(2):
      pltpu.async_copy(ones_v, acc.at[idx[j].at[u]], ssem[j], add=True)

  def wait_scatter(j):
    for u in range(2):
      pltpu.make_async_copy(ones_v, acc.at[idx[j].at[u]], ssem[j]).wait()

  fetch_idx(0, 0)
  fetch_idx(1, 1)
  wait_idx(0); scatter(0); fetch_idx(2, 2)
  wait_idx(1); scatter(1)

  def piped(i, carry):
    for u in range(3):
      t = 2 + i * 3 + u
      j = (2 + u) % 3
      jp = (j + 1) % 3
      wait_scatter(jp)
      fetch_idx(t + 1, jp)
      wait_idx(j)
      scatter(j)
    return carry

  lax.fori_loop(0, (_NSS - 3) // 3, piped, 0)

  _jl = (_NSS - 1) % 3
  wait_scatter((_jl + 1) % 3)
  wait_idx(_jl); scatter(_jl)
  wait_scatter((_jl + 2) % 3); wait_scatter(_jl)

  @pl.when(wid < _NCHUNK - 2 * _NSS * _NW)
  def _():
    cid = 2 * _NSS * _NW + wid
    pltpu.sync_copy(ei_hbm.at[1, pl.ds(cid, 1), :], idx[0].at[pl.ds(0, 1), :])
    pltpu.async_copy(ones_v, acc.at[idx[0].at[0]], ssem[0], add=True).wait()

  plsc.subcore_barrier()
  pltpu.sync_copy(
      acc.at[pl.ds(s * _RPT, _RPT)], out_hbm.at[c, pl.ds(s * _RPT, _RPT)]
  )


@functools.partial(
    pl.kernel,
    out_type=jax.ShapeDtypeStruct((_NC, _NP, _H), jnp.float32),
    mesh=_mesh,
    compiler_params=pltpu.CompilerParams(use_tc_tiling_on_sc=False),
    scratch_types=[
        [pltpu.VMEM((2, 2, _CHUNK), jnp.int32) for _ in range(3)],
        [pltpu.VMEM((2 * _CHUNK, _H), jnp.float32) for _ in range(3)],
        pltpu.VMEM((_ZR, _H), jnp.float32),
        pltpu.VMEM_SHARED((_NP, _H), jnp.float32),
        [pltpu.SemaphoreType.DMA for _ in range(3)],
        [pltpu.SemaphoreType.DMA for _ in range(3)],
        [pltpu.SemaphoreType.DMA for _ in range(3)],
    ],
)
def _conv_scatter(ei_hbm, hws_hbm, out_hbm, idx, rows, stage, acc, isem, gsem,
                  ssem):
  c = lax.axis_index("c")
  s = lax.axis_index("s")
  wid = s * _NC + c

  _zero_fill(stage, _ZR, _H)
  for i in range(_RPT // _ZR):
    pltpu.sync_copy(stage, acc.at[pl.ds(s * _RPT + i * _ZR, _ZR)])
  plsc.subcore_barrier()

  def fetch_idx(t, j):
    p = wid + t * _NW
    pltpu.async_copy(ei_hbm.at[:, pl.ds(2 * p, 2), :], idx[j], isem[j])

  def wait_idx(j):
    pltpu.make_async_copy(
        ei_hbm.at[:, pl.ds(0, 2), :], idx[j], isem[j]).wait()

  def gather(j):
    for u in range(2):
      pltpu.async_copy(hws_hbm.at[idx[j].at[0, u]],
                       rows[j].at[pl.ds(u * _CHUNK, _CHUNK)], gsem[j])

  def wait_gather(j):
    for u in range(2):
      pltpu.make_async_copy(hws_hbm.at[idx[j].at[0, u]],
                            rows[j].at[pl.ds(u * _CHUNK, _CHUNK)],
                            gsem[j]).wait()

  def scatter(j):
    for u in range(2):
      pltpu.async_copy(rows[j].at[pl.ds(u * _CHUNK, _CHUNK)],
                       acc.at[idx[j].at[1, u]], ssem[j], add=True)

  def wait_scatter(j):
    for u in range(2):
      pltpu.make_async_copy(rows[j].at[pl.ds(u * _CHUNK, _CHUNK)],
                            acc.at[idx[j].at[1, u]], ssem[j]).wait()

  fetch_idx(0, 0)
  fetch_idx(1, 1)
  wait_idx(0); gather(0); fetch_idx(2, 2)
  wait_idx(1); gather(1); wait_gather(0); scatter(0)

  def piped(i, carry):
    for u in range(3):
      t = 2 + i * 3 + u
      j = (2 + u) % 3
      jp = (j + 1) % 3
      jm = (j + 2) % 3
      wait_scatter(jp)
      fetch_idx(t + 1, jp)
      wait_idx(j)
      gather(j)
      wait_gather(jm)
      scatter(jm)
    return carry

  lax.fori_loop(0, (_NSS - 3) // 3, piped, 0)

  _jl = (_NSS - 1) % 3
  wait_scatter((_jl + 1) % 3)
  wait_idx(_jl); gather(_jl)
  wait_gather((_jl + 2) % 3); scatter((_jl + 2) % 3)
  wait_gather(_jl); scatter(_jl)
  wait_scatter((_jl + 2) % 3); wait_scatter(_jl)

  @pl.when(wid < _NCHUNK - 2 * _NSS * _NW)
  def _():
    cid = 2 * _NSS * _NW + wid
    pltpu.sync_copy(ei_hbm.at[:, pl.ds(cid, 1), :], idx[0].at[:, pl.ds(0, 1), :])
    pltpu.async_copy(hws_hbm.at[idx[0].at[0, 0]],
                     rows[0].at[pl.ds(0, _CHUNK)], gsem[0]).wait()
    pltpu.async_copy(rows[0].at[pl.ds(0, _CHUNK)],
                     acc.at[idx[0].at[1, 0]], ssem[0], add=True).wait()

  plsc.subcore_barrier()
  pltpu.sync_copy(
      acc.at[pl.ds(s * _RPT, _RPT)], out_hbm.at[c, pl.ds(s * _RPT, _RPT)]
  )


def _bn_stats(h):
  m = jnp.mean(h, axis=0, keepdims=True)
  v = jnp.mean((h - m) ** 2, axis=0, keepdims=True)
  return m, v


def _tc_prep(deg_ref, x_ref, w1_ref, sigma_ref, wf1_ref, bf1_ref, wf2_ref,
             bf2_ref, dinv_ref, hws1_ref, f_ref):
  deg = deg_ref[0, 0:_N, 0:1] + deg_ref[1, 0:_N, 0:1] + 1.0
  dinv = lax.rsqrt(deg)
  dinv_ref[...] = dinv
  hw = jnp.dot(x_ref[...], w1_ref[...], preferred_element_type=jnp.float32)
  hws1_ref[...] = hw * dinv
  f = jnp.maximum(
      jnp.dot(sigma_ref[...], wf1_ref[...], preferred_element_type=jnp.float32)
      + bf1_ref[...], 0.0)
  f_ref[...] = jnp.maximum(
      jnp.dot(f, wf2_ref[...], preferred_element_type=jnp.float32)
      + bf2_ref[...], 0.0)


def _tc_mid1(acc_ref, hws_ref, dinv_ref, b_ref, g_ref, be_ref, w2_ref,
             h1_ref, hws2_ref):
  dinv = dinv_ref[...]
  agg = (acc_ref[0, 0:_N] + acc_ref[1, 0:_N] + hws_ref[...]) * dinv + b_ref[...]
  m, v = _bn_stats(agg)
  h1 = jnp.maximum((agg - m) * lax.rsqrt(v + 1e-5) * g_ref[...] + be_ref[...], 0.0)
  h1_ref[...] = h1
  hws2_ref[...] = (
      jnp.dot(h1, w2_ref[...], preferred_element_type=jnp.float32) * dinv
  )


def _tc_mid2(acc_ref, hws_ref, dinv_ref, b_ref, g_ref, be_ref, h1_ref, w3_ref,
             hws3_ref):
  dinv = dinv_ref[...]
  agg = (acc_ref[0, 0:_N] + acc_ref[1, 0:_N] + hws_ref[...]) * dinv + b_ref[...]
  m, v = _bn_stats(agg)
  bn2 = (agg - m) * lax.rsqrt(v + 1e-5) * g_ref[...] + be_ref[...]
  h2 = jnp.maximum(bn2 + h1_ref[...], 0.0)
  hws3_ref[...] = (
      jnp.dot(h2, w3_ref[...], preferred_element_type=jnp.float32) * dinv
  )


def _tc_final(acc_ref, hws_ref, dinv_ref, b_ref, g_ref, be_ref, batch_ref,
              f_ref, wfc_ref, bfc_ref, out_ref):
  dinv = dinv_ref[...]
  agg = (acc_ref[0, 0:_N] + acc_ref[1, 0:_N] + hws_ref[...]) * dinv + b_ref[...]
  m, v = _bn_stats(agg)
  h3 = jnp.maximum((agg - m) * lax.rsqrt(v + 1e-5) * g_ref[...] + be_ref[...], 0.0)

  gids = lax.broadcasted_iota(jnp.int32, (_G, 1), 0)
  onehot = (batch_ref[...] == gids).astype(jnp.float32)
  sums = jnp.dot(onehot, h3, preferred_element_type=jnp.float32)
  cnt = jnp.sum(onehot, axis=1, keepdims=True)
  gemb = sums / jnp.maximum(cnt, 1.0)

  f = f_ref[...]

  out_ref[...] = (
      jnp.dot(gemb, wfc_ref[0:_H, :], preferred_element_type=jnp.float32)
      + jnp.dot(f, wfc_ref[_H:, :], preferred_element_type=jnp.float32)
      + bfc_ref[...]
  )


def kernel(x, edge_index, batch, sigma, W1, b1, W2, b2, W3, b3, g1, be1, g2,
           be2, g3, be3, Wf1, bf1, Wf2, bf2, Wfc, bfc):
  f32 = jnp.float32
  ei3 = edge_index.reshape(2, _NCHUNK, _CHUNK)

  degp = _deg_scatter(ei3)

  dinv, hws1, fbr = pl.pallas_call(
      _tc_prep,
      out_shape=[
          jax.ShapeDtypeStruct((_N, 1), f32),
          jax.ShapeDtypeStruct((_N, _H), f32),
          jax.ShapeDtypeStruct((_G, _H), f32),
      ],
  )(degp, x, W1, sigma, Wf1, bf1.reshape(1, 2 * _H), Wf2,
    bf2.reshape(1, _H))

  acc1 = _conv_scatter(ei3, hws1)

  h1, hws2 = pl.pallas_call(
      _tc_mid1,
      out_shape=[
          jax.ShapeDtypeStruct((_N, _H), f32),
          jax.ShapeDtypeStruct((_N, _H), f32),
      ],
  )(acc1, hws1, dinv, b1.reshape(1, _H), g1.reshape(1, _H),
    be1.reshape(1, _H), W2)

  acc2 = _conv_scatter(ei3, hws2)

  hws3, = pl.pallas_call(
      _tc_mid2,
      out_shape=[jax.ShapeDtypeStruct((_N, _H), f32)],
  )(acc2, hws2, dinv, b2.reshape(1, _H), g2.reshape(1, _H),
    be2.reshape(1, _H), h1, W3)

  acc3 = _conv_scatter(ei3, hws3)

  out2d = pl.pallas_call(
      _tc_final,
      out_shape=jax.ShapeDtypeStruct((_G, 1), f32),
  )(acc3, hws3, dinv, b3.reshape(1, _H), g3.reshape(1, _H),
    be3.reshape(1, _H), batch.reshape(1, _N), fbr, Wfc,
    bfc.reshape(1, 1))

  return out2d.reshape(_G)


# speedup vs baseline: 50.2330x; 1.2699x over previous
import functools

import jax
import jax.numpy as jnp
from jax import lax
from jax.experimental import pallas as pl
from jax.experimental.pallas import tpu as pltpu
from jax.experimental.pallas import tpu_sc as plsc

_N = 10000
_E = 320000
_D = 128
_H = 64
_G = 16

_NC = 2
_NS = 16
_NW = _NC * _NS
_CHUNK = 128
_NCHUNK = _E // _CHUNK
_RPT = 632
_NP = _RPT * _NS
_DEGW = 16
_NSS = 39
_ZR = 79

_mesh = plsc.VectorSubcoreMesh(
    core_axis_name="c", subcore_axis_name="s", num_cores=_NC, num_subcores=_NS
)


def _zero_fill(ref, rows, width):
  zv = jnp.zeros((16,), jnp.float32)
  nw = width // 16

  def body(i, carry):
    r = i // nw
    cs = (i % nw) * 16
    ref[r, pl.ds(cs, 16)] = zv
    return carry

  lax.fori_loop(0, rows * nw, body, 0)


@functools.partial(
    pl.kernel,
    out_type=jax.ShapeDtypeStruct((_NC, _NP, _DEGW), jnp.float32),
    mesh=_mesh,
    compiler_params=pltpu.CompilerParams(use_tc_tiling_on_sc=False),
    scratch_types=[
        pltpu.VMEM((_CHUNK, _DEGW), jnp.float32),
        pltpu.VMEM((_ZR, _DEGW), jnp.float32),
        [pltpu.VMEM((2, _CHUNK), jnp.int32) for _ in range(3)],
        pltpu.VMEM_SHARED((_NP, _DEGW), jnp.float32),
        [pltpu.SemaphoreType.DMA for _ in range(3)],
        [pltpu.SemaphoreType.DMA for _ in range(3)],
    ],
)
def _deg_scatter(ei_hbm, out_hbm, ones_v, stage, idx, acc, isem, ssem):
  c = lax.axis_index("c")
  s = lax.axis_index("s")
  wid = s * _NC + c

  ov = jnp.ones((16,), jnp.float32)

  def fill_ones(i, carry):
    r = i // (_DEGW // 16)
    cs = (i % (_DEGW // 16)) * 16
    ones_v[r, pl.ds(cs, 16)] = ov
    return carry

  lax.fori_loop(0, _CHUNK * (_DEGW // 16), fill_ones, 0)
  _zero_fill(stage, _ZR, _DEGW)
  for i in range(_RPT // _ZR):
    pltpu.sync_copy(stage, acc.at[pl.ds(s * _RPT + i * _ZR, _ZR)])
  plsc.subcore_barrier()

  def fetch_idx(t, j):
    p = wid + t * _NW
    pltpu.async_copy(ei_hbm.at[1, pl.ds(2 * p, 2), :], idx[j], isem[j])

  def wait_idx(j):
    pltpu.make_async_copy(ei_hbm.at[1, pl.ds(0, 2), :], idx[j], isem[j]).wait()

  def scatter(j):
    for u in range(2):
      pltpu.async_copy(ones_v, acc.at[idx[j].at[u]], ssem[j], add=True)

  def wait_scatter(j):
    for u in range(2):
      pltpu.make_async_copy(ones_v, acc.at[idx[j].at[u]], ssem[j]).wait()

  fetch_idx(0, 0)
  fetch_idx(1, 1)
  wait_idx(0); scatter(0); fetch_idx(2, 2)
  wait_idx(1); scatter(1)

  def piped(i, carry):
    for u in range(3):
      t = 2 + i * 3 + u
      j = (2 + u) % 3
      jp = (j + 1) % 3
      wait_scatter(jp)
      fetch_idx(t + 1, jp)
      wait_idx(j)
      scatter(j)
    return carry

  lax.fori_loop(0, (_NSS - 3) // 3, piped, 0)

  _jl = (_NSS - 1) % 3
  wait_scatter((_jl + 1) % 3)
  wait_idx(_jl); scatter(_jl)
  wait_scatter((_jl + 2) % 3); wait_scatter(_jl)

  @pl.when(wid < _NCHUNK - 2 * _NSS * _NW)
  def _():
    cid = 2 * _NSS * _NW + wid
    pltpu.sync_copy(ei_hbm.at[1, pl.ds(cid, 1), :], idx[0].at[pl.ds(0, 1), :])
    pltpu.async_copy(ones_v, acc.at[idx[0].at[0]], ssem[0], add=True).wait()

  plsc.subcore_barrier()
  pltpu.sync_copy(
      acc.at[pl.ds(s * _RPT, _RPT)], out_hbm.at[c, pl.ds(s * _RPT, _RPT)]
  )


@functools.partial(
    pl.kernel,
    out_type=jax.ShapeDtypeStruct((_NC, _NP, _H), jnp.float32),
    mesh=_mesh,
    compiler_params=pltpu.CompilerParams(use_tc_tiling_on_sc=False),
    scratch_types=[
        [pltpu.VMEM((2, 2, _CHUNK), jnp.int32) for _ in range(3)],
        [pltpu.VMEM((2 * _CHUNK, _H), jnp.float32) for _ in range(3)],
        pltpu.VMEM((_ZR, _H), jnp.float32),
        pltpu.VMEM_SHARED((_NP, _H), jnp.float32),
        [pltpu.SemaphoreType.DMA for _ in range(3)],
        [pltpu.SemaphoreType.DMA for _ in range(3)],
        [pltpu.SemaphoreType.DMA for _ in range(3)],
    ],
)
def _conv_scatter(ei_hbm, hws_hbm, out_hbm, idx, rows, stage, acc, isem, gsem,
                  ssem):
  c = lax.axis_index("c")
  s = lax.axis_index("s")
  wid = s * _NC + c

  _zero_fill(stage, _ZR, _H)
  for i in range(_RPT // _ZR):
    pltpu.sync_copy(stage, acc.at[pl.ds(s * _RPT + i * _ZR, _ZR)])
  plsc.subcore_barrier()

  def fetch_idx(t, j):
    p = wid + t * _NW
    pltpu.async_copy(ei_hbm.at[:, pl.ds(2 * p, 2), :], idx[j], isem[j])

  def wait_idx(j):
    pltpu.make_async_copy(
        ei_hbm.at[:, pl.ds(0, 2), :], idx[j], isem[j]).wait()

  def gather(j):
    for u in range(2):
      pltpu.async_copy(hws_hbm.at[idx[j].at[0, u]],
                       rows[j].at[pl.ds(u * _CHUNK, _CHUNK)], gsem[j])

  def wait_gather(j):
    for u in range(2):
      pltpu.make_async_copy(hws_hbm.at[idx[j].at[0, u]],
                            rows[j].at[pl.ds(u * _CHUNK, _CHUNK)],
                            gsem[j]).wait()

  def scatter(j):
    for u in range(2):
      pltpu.async_copy(rows[j].at[pl.ds(u * _CHUNK, _CHUNK)],
                       acc.at[idx[j].at[1, u]], ssem[j], add=True)

  def wait_scatter(j):
    for u in range(2):
      pltpu.make_async_copy(rows[j].at[pl.ds(u * _CHUNK, _CHUNK)],
                            acc.at[idx[j].at[1, u]], ssem[j]).wait()

  fetch_idx(0, 0)
  fetch_idx(1, 1)
  wait_idx(0); gather(0); fetch_idx(2, 2)
  wait_idx(1); gather(1); wait_gather(0); scatter(0)

  def piped(i, carry):
    for u in range(3):
      t = 2 + i * 3 + u
      j = (2 + u) % 3
      jp = (j + 1) % 3
      jm = (j + 2) % 3
      wait_scatter(jp)
      fetch_idx(t + 1, jp)
      wait_idx(j)
      gather(j)
      wait_gather(jm)
      scatter(jm)
    return carry

  lax.fori_loop(0, (_NSS - 3) // 3, piped, 0)

  _jl = (_NSS - 1) % 3
  wait_scatter((_jl + 1) % 3)
  wait_idx(_jl); gather(_jl)
  wait_gather((_jl + 2) % 3); scatter((_jl + 2) % 3)
  wait_gather(_jl); scatter(_jl)
  wait_scatter((_jl + 2) % 3); wait_scatter(_jl)

  @pl.when(wid < _NCHUNK - 2 * _NSS * _NW)
  def _():
    cid = 2 * _NSS * _NW + wid
    pltpu.sync_copy(ei_hbm.at[:, pl.ds(cid, 1), :], idx[0].at[:, pl.ds(0, 1), :])
    pltpu.async_copy(hws_hbm.at[idx[0].at[0, 0]],
                     rows[0].at[pl.ds(0, _CHUNK)], gsem[0]).wait()
    pltpu.async_copy(rows[0].at[pl.ds(0, _CHUNK)],
                     acc.at[idx[0].at[1, 0]], ssem[0], add=True).wait()

  plsc.subcore_barrier()
  pltpu.sync_copy(
      acc.at[pl.ds(s * _RPT, _RPT)], out_hbm.at[c, pl.ds(s * _RPT, _RPT)]
  )


def _bn_packed(h_pk, g2_ref, be2_ref):
  m_pk = jnp.mean(h_pk, axis=0, keepdims=True)
  m = 0.5 * (m_pk[:, 0:_H] + m_pk[:, _H:])
  mcc = jnp.concatenate([m, m], axis=1)
  e = h_pk - mcc
  v_pk = jnp.mean(e * e, axis=0, keepdims=True)
  v = 0.5 * (v_pk[:, 0:_H] + v_pk[:, _H:])
  vcc = jnp.concatenate([v, v], axis=1)
  return e * lax.rsqrt(vcc + 1e-5) * g2_ref[...] + be2_ref[...]


def _tc_mm1(x_ref, w1_ref, sigma_ref, wf1_ref, bf1_ref, wf2_ref, bf2_ref,
            hw1_ref, f_ref):
  hw1_ref[...] = jnp.dot(x_ref[...], w1_ref[...],
                         preferred_element_type=jnp.float32)
  f = jnp.maximum(
      jnp.dot(sigma_ref[...], wf1_ref[...], preferred_element_type=jnp.float32)
      + bf1_ref[...], 0.0)
  f_ref[...] = jnp.maximum(
      jnp.dot(f, wf2_ref[...], preferred_element_type=jnp.float32)
      + bf2_ref[...], 0.0)


def _tc_dinv(deg_ref, sel_ref, kexp_ref, dinv_ref):
  d16 = deg_ref[0] + deg_ref[1]
  a0 = jnp.dot(d16, sel_ref[...], preferred_element_type=jnp.float32) + 1.0
  r0 = lax.rsqrt(a0)
  dinv_ref[...] = jnp.dot(r0, kexp_ref[...],
                          preferred_element_type=jnp.float32)


def _tc_scale1(hw_ref, dinv_ref, hws1_ref):
  hws1_ref[...] = hw_ref[...] * dinv_ref[...]


def _tc_mid1(acc_ref, hws_ref, dinv_ref, b_ref, g_ref, be_ref, w2_ref,
             h1_ref, hws2_ref):
  dinv = dinv_ref[...]
  agg = (acc_ref[0, 0:_N // 2] + acc_ref[1, 0:_N // 2] + hws_ref[...]) * dinv \
      + b_ref[...]
  h1 = jnp.maximum(_bn_packed(agg, g_ref, be_ref), 0.0)
  h1_ref[...] = h1
  hws2_ref[...] = (
      jnp.dot(h1, w2_ref[...], preferred_element_type=jnp.float32) * dinv
  )


def _tc_mid2(acc_ref, hws_ref, dinv_ref, b_ref, g_ref, be_ref, h1_ref, w3_ref,
             hws3_ref):
  dinv = dinv_ref[...]
  agg = (acc_ref[0, 0:_N // 2] + acc_ref[1, 0:_N // 2] + hws_ref[...]) * dinv \
      + b_ref[...]
  h2 = jnp.maximum(_bn_packed(agg, g_ref, be_ref) + h1_ref[...], 0.0)
  hws3_ref[...] = (
      jnp.dot(h2, w3_ref[...], preferred_element_type=jnp.float32) * dinv
  )


def _tc_final(acc_ref, hws_ref, dinv_ref, b_ref, g_ref, be_ref, batch_ref,
              f_ref, wfc_ref, bfc_ref, out_ref):
  dinv = dinv_ref[...]
  agg = (acc_ref[0, 0:_N // 2] + acc_ref[1, 0:_N // 2] + hws_ref[...]) * dinv \
      + b_ref[...]
  h3 = jnp.maximum(_bn_packed(agg, g_ref, be_ref), 0.0)

  gids = lax.broadcasted_iota(jnp.int32, (_G, 1), 0)
  ohe = (batch_ref[0:1, :] == gids).astype(jnp.float32)
  oho = (batch_ref[1:2, :] == gids).astype(jnp.float32)
  sums = (
      jnp.dot(ohe, h3[:, 0:_H], preferred_element_type=jnp.float32)
      + jnp.dot(oho, h3[:, _H:], preferred_element_type=jnp.float32)
  )
  cnt = (jnp.sum(ohe, axis=1, keepdims=True)
         + jnp.sum(oho, axis=1, keepdims=True))
  gemb = sums / jnp.maximum(cnt, 1.0)

  f = f_ref[...]

  out_ref[...] = (
      jnp.dot(gemb, wfc_ref[0:_H, :], preferred_element_type=jnp.float32)
      + jnp.dot(f, wfc_ref[_H:, :], preferred_element_type=jnp.float32)
      + bfc_ref[...]
  )


def _blockdiag2(w):
  h_in, h_out = w.shape
  z = jnp.zeros((h_in, h_out), jnp.float32)
  return jnp.concatenate([
      jnp.concatenate([w, z], axis=1),
      jnp.concatenate([z, w], axis=1),
  ], axis=0)


def kernel(x, edge_index, batch, sigma, W1, b1, W2, b2, W3, b3, g1, be1, g2,
           be2, g3, be3, Wf1, bf1, Wf2, bf2, Wfc, bfc):
  f32 = jnp.float32
  npk = _N // 2
  ei3 = edge_index.reshape(2, _NCHUNK, _CHUNK)

  x_pk = x.reshape(npk, 2 * _D)
  w1s = _blockdiag2(W1)
  w2bd = _blockdiag2(W2)
  w3bd = _blockdiag2(W3)
  b1t = jnp.tile(b1.reshape(1, _H), (1, 2))
  b2t = jnp.tile(b2.reshape(1, _H), (1, 2))
  b3t = jnp.tile(b3.reshape(1, _H), (1, 2))
  g1t = jnp.tile(g1.reshape(1, _H), (1, 2))
  g2t = jnp.tile(g2.reshape(1, _H), (1, 2))
  g3t = jnp.tile(g3.reshape(1, _H), (1, 2))
  be1t = jnp.tile(be1.reshape(1, _H), (1, 2))
  be2t = jnp.tile(be2.reshape(1, _H), (1, 2))
  be3t = jnp.tile(be3.reshape(1, _H), (1, 2))
  ci = jnp.arange(128)[:, None]
  ki = jnp.arange(8)[None, :]
  sel = (ci == ki * _DEGW).astype(f32)
  kexp = jnp.kron(jnp.eye(8, dtype=f32), jnp.ones((1, _H), f32))
  batch2 = batch.reshape(npk, 2).transpose(1, 0)

  degp = _deg_scatter(ei3)
  degp_pk = degp.reshape(2, _NP * _DEGW // 128, 128)

  hw1, fbr = pl.pallas_call(
      _tc_mm1,
      out_shape=[
          jax.ShapeDtypeStruct((npk, 2 * _H), f32),
          jax.ShapeDtypeStruct((_G, _H), f32),
      ],
  )(x_pk, w1s, sigma, Wf1, bf1.reshape(1, 2 * _H), Wf2, bf2.reshape(1, _H))

  dinv512 = pl.pallas_call(
      _tc_dinv,
      out_shape=jax.ShapeDtypeStruct((_NP // 8, 8 * _H), f32),
  )(degp_pk, sel, kexp)
  dinv = dinv512.reshape(_NP // 2, 2 * _H)[0:npk]

  hws1 = pl.pallas_call(
      _tc_scale1,
      out_shape=jax.ShapeDtypeStruct((npk, 2 * _H), f32),
  )(hw1, dinv)

  acc1 = _conv_scatter(ei3, hws1.reshape(_N, _H)).reshape(2, _NP // 2, 2 * _H)

  h1, hws2 = pl.pallas_call(
      _tc_mid1,
      out_shape=[
          jax.ShapeDtypeStruct((npk, 2 * _H), f32),
          jax.ShapeDtypeStruct((npk, 2 * _H), f32),
      ],
  )(acc1, hws1, dinv, b1t, g1t, be1t, w2bd)

  acc2 = _conv_scatter(ei3, hws2.reshape(_N, _H)).reshape(2, _NP // 2, 2 * _H)

  hws3, = pl.pallas_call(
      _tc_mid2,
      out_shape=[jax.ShapeDtypeStruct((npk, 2 * _H), f32)],
  )(acc2, hws2, dinv, b2t, g2t, be2t, h1, w3bd)

  acc3 = _conv_scatter(ei3, hws3.reshape(_N, _H)).reshape(2, _NP // 2, 2 * _H)

  out2d = pl.pallas_call(
      _tc_final,
      out_shape=jax.ShapeDtypeStruct((_G, 1), f32),
  )(acc3, hws3, dinv, b3t, g3t, be3t, batch2, fbr, Wfc,
    bfc.reshape(1, 1))

  return out2d.reshape(_G)
